# Initial kernel scaffold; baseline (speedup 1.0000x reference)
import jax
import jax.numpy as jnp
from jax.experimental import pallas as pl


def kernel(x, edge_index, W, b):
    raise NotImplementedError("write your pallas kernel here")


# speedup vs baseline: 16.0526x; 16.0526x over previous
import functools

import jax
import jax.numpy as jnp
from jax import lax
from jax.experimental import pallas as pl
from jax.experimental.pallas import tpu as pltpu
from jax.experimental.pallas import tpu_sc as plsc

N = 10000
E = 320000
D = 128

NC = 2
NS = 16
NW = NC * NS
L = 16

EPW_RAW = E // NW
CH = (EPW_RAW + 127) // 128
EPW = CH * 128
NPAD = EPW
RPT = NPAD // NS

_mesh = plsc.VectorSubcoreMesh(core_axis_name="c", subcore_axis_name="s")
_sc_params = pltpu.CompilerParams(needs_layout_passes=False)


@functools.partial(
    pl.kernel,
    mesh=_mesh,
    out_type=jax.ShapeDtypeStruct((NW, NPAD), jnp.float32),
    compiler_params=_sc_params,
    scratch_types=[
        pltpu.VMEM((EPW,), jnp.int32),
        pltpu.VMEM((NPAD,), jnp.float32),
    ],
)
def _deg_kernel(dst_hbm, out_hbm, d_v, hist_v):
    cid = lax.axis_index("c")
    sid = lax.axis_index("s")
    wid = sid * NC + cid
    pltpu.sync_copy(dst_hbm.at[wid], d_v)
    zeros16 = jnp.zeros((L,), jnp.float32)
    ones16 = jnp.ones((L,), jnp.float32)

    def zbody(i, c):
        hist_v[pl.ds(i * L, L)] = zeros16
        return c

    lax.fori_loop(0, NPAD // L, zbody, 0)

    def body(i, c):
        d = d_v[pl.ds(i * L, L)]
        plsc.addupdate_scatter(hist_v, [d], ones16)
        return c

    lax.fori_loop(0, EPW // L, body, 0)
    pltpu.sync_copy(hist_v, out_hbm.at[wid])


@functools.partial(
    pl.kernel,
    mesh=_mesh,
    out_type=jax.ShapeDtypeStruct((NC, NPAD, D), jnp.float32),
    compiler_params=_sc_params,
    scratch_types=[
        pltpu.VMEM((CH, 128), jnp.int32),
        pltpu.VMEM((CH, 128), jnp.int32),
        pltpu.VMEM((128, D), jnp.float32),
        pltpu.VMEM_SHARED((NPAD, D), jnp.float32),
        pltpu.SemaphoreType.DMA,
    ],
)
def _scatter_kernel(p_hbm, src_hbm, dst_hbm, z_hbm, out_hbm,
                    si_v, di_v, rows_v, s_sh, sem):
    cid = lax.axis_index("c")
    sid = lax.axis_index("s")
    wid = sid * NC + cid
    pltpu.sync_copy(src_hbm.at[wid], si_v)
    pltpu.sync_copy(dst_hbm.at[wid], di_v)
    r0 = sid * RPT
    pltpu.sync_copy(z_hbm.at[pl.ds(r0, RPT)], s_sh.at[pl.ds(r0, RPT)])
    plsc.subcore_barrier()

    def body(j, c):
        pltpu.async_copy(p_hbm.at[si_v.at[j]], rows_v, sem).wait()
        pltpu.sync_copy(rows_v, s_sh.at[di_v.at[j]], add=True)
        return c

    lax.fori_loop(0, CH, body, 0)
    plsc.subcore_barrier()
    pltpu.sync_copy(s_sh.at[pl.ds(r0, RPT)], out_hbm.at[cid, pl.ds(r0, RPT)])


def _mm_body(x_ref, w_ref, parts_ref, p_ref):
    ones = jnp.ones((NW, 128), jnp.float32)
    deg = lax.dot_general(parts_ref[...], ones, (((0,), (0,)), ((), ())),
                          preferred_element_type=jnp.float32)
    dinv = lax.rsqrt(deg + 1.0)
    h = jnp.dot(x_ref[...], w_ref[...], preferred_element_type=jnp.float32)
    p_ref[...] = h * dinv


_mm_kernel = pl.pallas_call(
    _mm_body,
    grid=(NPAD // 128,),
    in_specs=[
        pl.BlockSpec((128, D), lambda i: (i, 0)),
        pl.BlockSpec((D, D), lambda i: (0, 0)),
        pl.BlockSpec((NW, 128), lambda i: (0, i)),
    ],
    out_specs=pl.BlockSpec((128, D), lambda i: (i, 0)),
    out_shape=jax.ShapeDtypeStruct((NPAD, D), jnp.float32),
)


def _fin_body(s_ref, p_ref, parts_ref, x_ref, b_ref, o_ref):
    ones = jnp.ones((NW, 128), jnp.float32)
    deg = lax.dot_general(parts_ref[...], ones, (((0,), (0,)), ((), ())),
                          preferred_element_type=jnp.float32)
    dinv = lax.rsqrt(deg + 1.0)
    agg = dinv * (s_ref[0] + s_ref[1] + p_ref[...]) + b_ref[...]
    o_ref[...] = jnp.maximum(agg, 0.0) + x_ref[...]


_fin_kernel = pl.pallas_call(
    _fin_body,
    grid=(NPAD // 128,),
    in_specs=[
        pl.BlockSpec((NC, 128, D), lambda i: (0, i, 0)),
        pl.BlockSpec((128, D), lambda i: (i, 0)),
        pl.BlockSpec((NW, 128), lambda i: (0, i)),
        pl.BlockSpec((128, D), lambda i: (i, 0)),
        pl.BlockSpec((1, D), lambda i: (0, 0)),
    ],
    out_specs=pl.BlockSpec((128, D), lambda i: (i, 0)),
    out_shape=jax.ShapeDtypeStruct((NPAD, D), jnp.float32),
)


def kernel(x, edge_index, W, b):
    src = edge_index[0].reshape(NW, EPW_RAW)
    dst = edge_index[1].reshape(NW, EPW_RAW)
    pad = EPW - EPW_RAW
    srcp = jnp.pad(src, ((0, 0), (0, pad)))
    dstp = jnp.pad(dst, ((0, 0), (0, pad)), constant_values=N)
    xp = jnp.pad(x, ((0, NPAD - N), (0, 0)))

    parts = _deg_kernel(dstp)
    p = _mm_kernel(xp, W, parts)
    zeros = jnp.zeros((NPAD, D), jnp.float32)
    s = _scatter_kernel(p, srcp.reshape(NW, CH, 128),
                        dstp.reshape(NW, CH, 128), zeros)
    out = _fin_kernel(s, p, parts, xp, b.reshape(1, D))
    return out[:N]
